# baseline (speedup 1.0000x reference)
import math

import jax
import jax.numpy as jnp
from jax.experimental import pallas as pl

_B, _N, _CIN, _COUT, _KNN, _R, _H, _W, _FS = 4, 4096, 32, 32, 16, 8, 260, 346, 3


_SEL_QB = 128


def _make_select_kernel(K, np_full, qb, rr):
    ball = rr is not None

    def kern(q_ref, p_ref, out_ref):
        q = q_ref[0]
        pt = p_ref[0]
        dot = jax.lax.dot_general(
            q, pt, (((1,), (0,)), ((), ())),
            preferred_element_type=jnp.float32)
        q2 = jnp.sum(q * q, axis=1, keepdims=True)
        p2 = jnp.sum(pt * pt, axis=0, keepdims=True)
        d2 = q2 + p2 - 2.0 * dot
        ci = jax.lax.broadcasted_iota(jnp.int32, (qb, np_full), 1)
        cols = []
        if ball:
            rows = (jax.lax.broadcasted_iota(jnp.int32, (qb, 1), 0)
                    + pl.program_id(1) * qb)
            keys = jnp.where(d2 <= rr, ci, np_full)
            kprev = jnp.full((qb, 1), -1, jnp.int32)
            for _ in range(K):
                cand = jnp.where(keys > kprev, keys, np_full)
                m = jnp.min(cand, axis=1, keepdims=True)
                cols.append(jnp.where(m < np_full, m, rows))
                kprev = m
        else:
            mprev = jnp.full((qb, 1), -jnp.inf, jnp.float32)
            iprev = jnp.full((qb, 1), -1, jnp.int32)
            for _ in range(K):
                gt = (d2 > mprev) | ((d2 == mprev) & (ci > iprev))
                cand = jnp.where(gt, d2, jnp.inf)
                m = jnp.min(cand, axis=1, keepdims=True)
                isel = jnp.min(jnp.where(cand == m, ci, np_full), axis=1,
                               keepdims=True)
                cols.append(isel)
                mprev, iprev = m, isel
        out_ref[0] = jnp.concatenate(cols, axis=1)

    return kern


def _select_k(q, p, K, rr=None):
    b, nq, d = q.shape
    np_full = p.shape[1]
    qb = min(_SEL_QB, nq)
    pt = jnp.swapaxes(p, 1, 2)
    grid = (b, nq // qb)
    return pl.pallas_call(
        _make_select_kernel(K, np_full, qb, rr),
        grid=grid,
        in_specs=[
            pl.BlockSpec((1, qb, d), lambda bi, i: (bi, i, 0)),
            pl.BlockSpec((1, d, np_full), lambda bi, i: (bi, 0, 0)),
        ],
        out_specs=pl.BlockSpec((1, qb, K), lambda bi, i: (bi, i, 0)),
        out_shape=jax.ShapeDtypeStruct((b, nq, K), jnp.int32),
    )(q, pt)


def _knn_idx(q, p, K):
    return _select_k(q, p, K)


def _knn_gather(x, idx):
    b, nq, k = idx.shape
    out = jnp.take_along_axis(x, idx.reshape(b, nq * k, 1), axis=1)
    return out.reshape(b, nq, k, x.shape[-1])


def _make_fps_kernel(n, ks):
    def kern(x_ref, out_ref):
        x = x_ref[0]
        ci = jax.lax.broadcasted_iota(jnp.int32, (1, n), 1)
        ck = jax.lax.broadcasted_iota(jnp.int32, (1, ks), 1)

        def body(i, st):
            dmin, last, idxs = st
            diff = x - last
            d = jnp.sum(diff * diff, axis=0, keepdims=True)
            dmin = jnp.minimum(dmin, d)
            m = jnp.max(dmin)
            nxt = jnp.min(jnp.where(dmin == m, ci, n))
            idxs = jnp.where(ck == i, nxt, idxs)
            last = jnp.sum(jnp.where(ci == nxt, x, 0.0), axis=1,
                           keepdims=True)
            return (dmin, last, idxs)

        dmin0 = jnp.full((1, n), jnp.inf, jnp.float32)
        last0 = x[:, 0:1]
        idxs0 = jnp.zeros((1, ks), jnp.int32)
        _, _, idxs = jax.lax.fori_loop(1, ks, body, (dmin0, last0, idxs0))
        out_ref[0] = idxs

    return kern


def _fps(x, K):
    b, n, _ = x.shape
    xt = jnp.swapaxes(x, 1, 2)
    xt = jnp.concatenate(
        [xt, jnp.zeros((b, 5, n), xt.dtype)], axis=1)
    return pl.pallas_call(
        _make_fps_kernel(n, K),
        grid=(b,),
        in_specs=[pl.BlockSpec((1, 8, n), lambda bi: (bi, 0, 0))],
        out_specs=pl.BlockSpec((1, 1, K), lambda bi: (bi, 0, 0)),
        out_shape=jax.ShapeDtypeStruct((b, 1, K), jnp.int32),
    )(xt)[:, 0, :]


def _pos_encoder(x, p):
    x = x @ p['w1'].T + p['b1']
    b, n, k, f = x.shape
    xr = x.reshape(b * n, k, f)
    mean = jnp.mean(xr, axis=(0, 1))
    var = jnp.var(xr, axis=(0, 1))
    xr = (xr - mean) * jax.lax.rsqrt(var + 1e-5) * p['g'] + p['be']
    xr = jax.nn.relu(xr)
    xr = xr @ p['w2'].T + p['b2']
    return xr.reshape(b, n, k, -1)


_ATTN_NB = 512


def _attn_kernel(varphi_ref, psi_ref, alpha_ref, delta_ref, g_ref, b_ref,
                 out_ref):
    scale = math.sqrt(_COUT)
    varphi = varphi_ref[0]
    psi = psi_ref[0]
    alpha = alpha_ref[0]
    delta = delta_ref[0]
    x = varphi[:, None, :] - psi + delta
    mu = jnp.mean(x, axis=-1, keepdims=True)
    xc = x - mu
    v = jnp.mean(xc * xc, axis=-1, keepdims=True)
    ln = xc * jax.lax.rsqrt(v + 1e-5) * g_ref[:] + b_ref[:]
    ln = ln / scale
    m = jnp.max(ln, axis=1, keepdims=True)
    e = jnp.exp(ln - m)
    a = e / jnp.sum(e, axis=1, keepdims=True)
    out_ref[0] = jnp.sum(a * (alpha + delta), axis=1)


def _attn_block(varphi, psi, alpha, delta, ln_g, ln_b):
    b, n, k, c = psi.shape
    nb = _ATTN_NB
    grid = (b, n // nb)
    return pl.pallas_call(
        _attn_kernel,
        grid=grid,
        in_specs=[
            pl.BlockSpec((1, nb, c), lambda bi, i: (bi, i, 0)),
            pl.BlockSpec((1, nb, k, c), lambda bi, i: (bi, i, 0, 0)),
            pl.BlockSpec((1, nb, k, c), lambda bi, i: (bi, i, 0, 0)),
            pl.BlockSpec((1, nb, k, c), lambda bi, i: (bi, i, 0, 0)),
            pl.BlockSpec((c,), lambda bi, i: (0,)),
            pl.BlockSpec((c,), lambda bi, i: (0,)),
        ],
        out_specs=pl.BlockSpec((1, nb, c), lambda bi, i: (bi, i, 0)),
        out_shape=jax.ShapeDtypeStruct((b, n, c), varphi.dtype),
    )(varphi, psi, alpha, delta, ln_g, ln_b)


def _lxformer(xytp, features, p):
    xyt = jax.lax.stop_gradient(xytp[:, :, :3])
    idx = _knn_idx(xyt, xyt, _KNN)
    delta = _pos_encoder(xytp[:, :, None, :] - _knn_gather(xytp, idx), p['pe'])
    t = features @ p['tw'].T + p['tb']
    c = t.shape[-1] // 3
    varphi, psi, alpha = t[..., :c], t[..., c:2 * c], t[..., 2 * c:]
    psi = _knn_gather(psi, idx)
    alpha = _knn_gather(alpha, idx)
    return _attn_block(varphi, psi, alpha, delta, p['ln_g'], p['ln_b'])


def _sparse_conv(xytp, features, cw, cb):
    b, n = xytp.shape[:2]
    pos = xytp[..., 3:4]
    neg = 1.0 - pos
    inp = jnp.concatenate([pos, neg, features], axis=-1)
    yy = jnp.clip(jnp.round(xytp[..., 2] * _H).astype(jnp.int32), 0, _H - 1)
    xx = jnp.clip(jnp.round(xytp[..., 1] * _W).astype(jnp.int32), 0, _W - 1)
    grid = jnp.zeros((b, _H, _W, inp.shape[-1]), dtype=inp.dtype)
    bidx = jnp.broadcast_to(jnp.arange(b)[:, None], (b, n))
    grid = grid.at[bidx, yy, xx].add(inp)
    out = jax.lax.conv_general_dilated(
        grid, cw, (1, 1), 'SAME',
        dimension_numbers=('NHWC', 'HWIO', 'NHWC')) + cb
    return out[bidx, yy, xx]


def _scformer(xytp, features, p):
    xyt = jax.lax.stop_gradient(xytp[..., :3])
    xyt = xyt.at[..., 0].set(0.0)
    radius = 5.0 / _H
    idx = _select_k(xyt, xyt, _KNN, rr=radius * radius)
    xy = xytp[..., jnp.array([1, 2])]
    delta = _pos_encoder(xy[:, :, None, :] - _knn_gather(xy, idx), p['pe'])
    t = _sparse_conv(xytp, features, p['cw'], p['cb'])
    c = t.shape[-1] // 3
    varphi, psi, alpha = t[..., :c], t[..., c:2 * c], t[..., 2 * c:]
    psi = _knn_gather(psi, idx)
    alpha = _knn_gather(alpha, idx)
    return _attn_block(varphi, psi, alpha, delta, p['ln_g'], p['ln_b'])


def _gxformer(xytp, features, p):
    xyt = jax.lax.stop_gradient(xytp[:, :, :3])
    Ks = xytp.shape[1] // _R
    sample_idx = jax.lax.stop_gradient(_fps(xyt, Ks))
    sample_xyt = jnp.take_along_axis(xyt, sample_idx[:, :, None], axis=1)
    sample_xytp = _knn_gather(xytp, sample_idx[:, :, None])[:, :, 0, :]
    pair_idx = _knn_idx(sample_xyt, xyt, _KNN)
    inv_pair_idx = _knn_idx(xyt, sample_xyt, _KNN)
    delta = _pos_encoder(
        xytp[:, :, None, :] - _knn_gather(sample_xytp, inv_pair_idx), p['pe'])
    t = features @ p['tw'].T + p['tb']
    c = t.shape[-1] // 3
    varphi, psi, alpha = t[..., :c], t[..., c:2 * c], t[..., 2 * c:]
    psi = _knn_gather(psi, pair_idx)
    alpha = _knn_gather(alpha, pair_idx)
    psi = jnp.max(psi, axis=2)
    alpha = jnp.max(alpha, axis=2)
    psi = _knn_gather(psi, inv_pair_idx)
    alpha = _knn_gather(alpha, inv_pair_idx)
    return _attn_block(varphi, psi, alpha, delta, p['ln_g'], p['ln_b'])


def kernel(xytp, features, params):
    lx = _lxformer(xytp, features, params['lx'])
    sc = _scformer(xytp, features, params['sc'])
    gx = lx
    h = jnp.concatenate([lx, sc, gx], axis=-1)
    h = h @ params['pw1'].T + params['pb1']
    h = jax.nn.gelu(h, approximate=False)
    return h @ params['pw2'].T + params['pb2']


# speedup vs baseline: 1.2746x; 1.2746x over previous
import math

import jax
import jax.numpy as jnp
from jax.experimental import pallas as pl

_B, _N, _CIN, _COUT, _KNN, _R, _H, _W, _FS = 4, 4096, 32, 32, 16, 8, 260, 346, 3


_SEL_QB = 128


def _make_select_kernel(K, np_full, qb, rr):
    ball = rr is not None

    def kern(q_ref, p_ref, out_ref):
        q = q_ref[0]
        pt = p_ref[0]
        dot = jax.lax.dot_general(
            q, pt, (((1,), (0,)), ((), ())),
            preferred_element_type=jnp.float32)
        q2 = jnp.sum(q * q, axis=1, keepdims=True)
        p2 = jnp.sum(pt * pt, axis=0, keepdims=True)
        d2 = q2 + p2 - 2.0 * dot
        ci = jax.lax.broadcasted_iota(jnp.int32, (qb, np_full), 1)
        cols = []
        if ball:
            rows = (jax.lax.broadcasted_iota(jnp.int32, (qb, 1), 0)
                    + pl.program_id(1) * qb)
            keys = jnp.where(d2 <= rr, ci, np_full)
            kprev = jnp.full((qb, 1), -1, jnp.int32)
            for _ in range(K):
                cand = jnp.where(keys > kprev, keys, np_full)
                m = jnp.min(cand, axis=1, keepdims=True)
                cols.append(jnp.where(m < np_full, m, rows))
                kprev = m
        else:
            mprev = jnp.full((qb, 1), -jnp.inf, jnp.float32)
            iprev = jnp.full((qb, 1), -1, jnp.int32)
            for _ in range(K):
                gt = (d2 > mprev) | ((d2 == mprev) & (ci > iprev))
                cand = jnp.where(gt, d2, jnp.inf)
                m = jnp.min(cand, axis=1, keepdims=True)
                isel = jnp.min(jnp.where(cand == m, ci, np_full), axis=1,
                               keepdims=True)
                cols.append(isel)
                mprev, iprev = m, isel
        out_ref[0] = jnp.concatenate(cols, axis=1)

    return kern


def _select_k(q, p, K, rr=None):
    b, nq, d = q.shape
    np_full = p.shape[1]
    qb = min(_SEL_QB, nq)
    pt = jnp.swapaxes(p, 1, 2)
    grid = (b, nq // qb)
    return pl.pallas_call(
        _make_select_kernel(K, np_full, qb, rr),
        grid=grid,
        in_specs=[
            pl.BlockSpec((1, qb, d), lambda bi, i: (bi, i, 0)),
            pl.BlockSpec((1, d, np_full), lambda bi, i: (bi, 0, 0)),
        ],
        out_specs=pl.BlockSpec((1, qb, K), lambda bi, i: (bi, i, 0)),
        out_shape=jax.ShapeDtypeStruct((b, nq, K), jnp.int32),
    )(q, pt)


def _knn_idx(q, p, K):
    return _select_k(q, p, K)


def _make_fps_kernel(n, ks):
    def kern(x_ref, out_ref):
        x = x_ref[0]
        ci = jax.lax.broadcasted_iota(jnp.int32, (1, n), 1)
        ck = jax.lax.broadcasted_iota(jnp.int32, (1, ks), 1)

        def body(i, st):
            dmin, last, idxs = st
            diff = x - last
            d = jnp.sum(diff * diff, axis=0, keepdims=True)
            dmin = jnp.minimum(dmin, d)
            m = jnp.max(dmin)
            nxt = jnp.min(jnp.where(dmin == m, ci, n))
            idxs = jnp.where(ck == i, nxt, idxs)
            last = jnp.sum(jnp.where(ci == nxt, x, 0.0), axis=1,
                           keepdims=True)
            return (dmin, last, idxs)

        dmin0 = jnp.full((1, n), jnp.inf, jnp.float32)
        last0 = x[:, 0:1]
        idxs0 = jnp.zeros((1, ks), jnp.int32)
        _, _, idxs = jax.lax.fori_loop(1, ks, body, (dmin0, last0, idxs0))
        out_ref[0] = idxs

    return kern


def _fps(x, K):
    b, n, _ = x.shape
    xt = jnp.swapaxes(x, 1, 2)
    xt = jnp.concatenate(
        [xt, jnp.zeros((b, 5, n), xt.dtype)], axis=1)
    return pl.pallas_call(
        _make_fps_kernel(n, K),
        grid=(b,),
        in_specs=[pl.BlockSpec((1, 8, n), lambda bi: (bi, 0, 0))],
        out_specs=pl.BlockSpec((1, 1, K), lambda bi: (bi, 0, 0)),
        out_shape=jax.ShapeDtypeStruct((b, 1, K), jnp.int32),
    )(xt)[:, 0, :]


_GB_QB = 128
_GB_CH = 512


def _onehot_gather(idx2, src, ncols, np_full):
    qbk = idx2.shape[0]
    ch = min(_GB_CH, np_full)
    acc = jnp.zeros((qbk, ncols), jnp.float32)
    for c in range(np_full // ch):
        ci = jax.lax.broadcasted_iota(jnp.int32, (qbk, ch), 1) + c * ch
        oh = (idx2 == ci).astype(jnp.float32)
        acc = acc + jax.lax.dot(oh, src[c * ch:(c + 1) * ch, :],
                                precision=jax.lax.Precision.HIGHEST,
                                preferred_element_type=jnp.float32)
    return acc


def _make_h1_stats_kernel(np_full, f):
    def kern(idxf_ref, qrep_ref, pc_ref, w1t_ref, b1_ref, h1_ref, ss_ref):
        idx2 = idxf_ref[0]
        g = _onehot_gather(idx2, pc_ref[0], f, np_full)
        diff = qrep_ref[0] - g
        h1 = jax.lax.dot(diff, w1t_ref[:],
                         preferred_element_type=jnp.float32) + b1_ref[:]
        h1_ref[0] = h1
        s0 = jnp.sum(h1, axis=0, keepdims=True)
        s1 = jnp.sum(h1 * h1, axis=0, keepdims=True)
        ss_ref[0, 0] = jnp.concatenate([s0, s1], axis=0)

    return kern


def _h1_stats(idx, qc, pc, pe):
    b, n, k = idx.shape
    np_full, f = pc.shape[1], pc.shape[2]
    nk = n * k
    qbk = _GB_QB * k
    nblk = n // _GB_QB
    idxf = idx.reshape(b, nk, 1)
    qrep = jnp.broadcast_to(qc[:, :, None, :], (b, n, k, f)).reshape(b, nk, f)
    h1, ss = pl.pallas_call(
        _make_h1_stats_kernel(np_full, f),
        grid=(b, nblk),
        in_specs=[
            pl.BlockSpec((1, qbk, 1), lambda bi, i: (bi, i, 0)),
            pl.BlockSpec((1, qbk, f), lambda bi, i: (bi, i, 0)),
            pl.BlockSpec((1, np_full, f), lambda bi, i: (bi, 0, 0)),
            pl.BlockSpec((f, f), lambda bi, i: (0, 0)),
            pl.BlockSpec((1, f), lambda bi, i: (0, 0)),
        ],
        out_specs=[
            pl.BlockSpec((1, qbk, f), lambda bi, i: (bi, i, 0)),
            pl.BlockSpec((1, 1, 2, f), lambda bi, i: (bi, i, 0, 0)),
        ],
        out_shape=[
            jax.ShapeDtypeStruct((b, nk, f), jnp.float32),
            jax.ShapeDtypeStruct((b, nblk, 2, f), jnp.float32),
        ],
    )(idxf, qrep, pc, pe['w1'].T, pe['b1'][None, :])
    s = jnp.sum(ss, axis=(0, 1))
    cnt = b * nk
    mean = s[0] / cnt
    var = s[1] / cnt - mean * mean
    return h1, mean[None, :], var[None, :]


def _make_branch_attn_kernel(np_full, f, k, qb):
    scale = math.sqrt(_COUT)
    c = _COUT

    def kern(idxf_ref, tq_ref, tsrc_ref, h1_ref, mean_ref, var_ref,
             g_ref, be_ref, w2t_ref, b2_ref, lng_ref, lnb_ref, out_ref):
        idx2 = idxf_ref[0]
        g64 = _onehot_gather(idx2, tsrc_ref[0], 2 * c, np_full)
        h1 = h1_ref[0]
        h1n = ((h1 - mean_ref[:]) * jax.lax.rsqrt(var_ref[:] + 1e-5)
               * g_ref[:] + be_ref[:])
        h1n = jnp.maximum(h1n, 0.0)
        delta = jax.lax.dot(h1n, w2t_ref[:],
                            preferred_element_type=jnp.float32) + b2_ref[:]
        psi = g64[:, :c].reshape(qb, k, c)
        alpha = g64[:, c:].reshape(qb, k, c)
        delta3 = delta.reshape(qb, k, c)
        varphi = tq_ref[0]
        x = varphi[:, None, :] - psi + delta3
        mu = jnp.mean(x, axis=-1, keepdims=True)
        xc = x - mu
        v = jnp.mean(xc * xc, axis=-1, keepdims=True)
        ln = (xc * jax.lax.rsqrt(v + 1e-5) * lng_ref[:] + lnb_ref[:]) / scale
        m = jnp.max(ln, axis=1, keepdims=True)
        e = jnp.exp(ln - m)
        a = e / jnp.sum(e, axis=1, keepdims=True)
        out_ref[0] = jnp.sum(a * (alpha + delta3), axis=1)

    return kern


def _branch_attn(idx, tq, tsrc, h1, mean, var, pe, ln_g, ln_b):
    b, n, k = idx.shape
    np_full = tsrc.shape[1]
    f = h1.shape[-1]
    c = _COUT
    qbk = _GB_QB * k
    nblk = n // _GB_QB
    idxf = idx.reshape(b, n * k, 1)
    return pl.pallas_call(
        _make_branch_attn_kernel(np_full, f, k, _GB_QB),
        grid=(b, nblk),
        in_specs=[
            pl.BlockSpec((1, qbk, 1), lambda bi, i: (bi, i, 0)),
            pl.BlockSpec((1, _GB_QB, c), lambda bi, i: (bi, i, 0)),
            pl.BlockSpec((1, np_full, 2 * c), lambda bi, i: (bi, 0, 0)),
            pl.BlockSpec((1, qbk, f), lambda bi, i: (bi, i, 0)),
            pl.BlockSpec((1, f), lambda bi, i: (0, 0)),
            pl.BlockSpec((1, f), lambda bi, i: (0, 0)),
            pl.BlockSpec((1, f), lambda bi, i: (0, 0)),
            pl.BlockSpec((1, f), lambda bi, i: (0, 0)),
            pl.BlockSpec((f, c), lambda bi, i: (0, 0)),
            pl.BlockSpec((1, c), lambda bi, i: (0, 0)),
            pl.BlockSpec((1, c), lambda bi, i: (0, 0)),
            pl.BlockSpec((1, c), lambda bi, i: (0, 0)),
        ],
        out_specs=pl.BlockSpec((1, _GB_QB, c), lambda bi, i: (bi, i, 0)),
        out_shape=jax.ShapeDtypeStruct((b, n, c), jnp.float32),
    )(idxf, tq, tsrc, h1, mean, var, pe['g'][None, :], pe['be'][None, :],
      pe['w2'].T, pe['b2'][None, :], ln_g[None, :], ln_b[None, :])


def _make_pool_kernel(np_full, k, qb):
    def kern(idxf_ref, tsrc_ref, out_ref):
        idx2 = idxf_ref[0]
        g64 = _onehot_gather(idx2, tsrc_ref[0], 2 * _COUT, np_full)
        out_ref[0] = jnp.max(g64.reshape(qb, k, 2 * _COUT), axis=1)

    return kern


def _pool(idx, tsrc):
    b, nq, k = idx.shape
    np_full = tsrc.shape[1]
    qb = min(_GB_QB, nq)
    qbk = qb * k
    idxf = idx.reshape(b, nq * k, 1)
    return pl.pallas_call(
        _make_pool_kernel(np_full, k, qb),
        grid=(b, nq // qb),
        in_specs=[
            pl.BlockSpec((1, qbk, 1), lambda bi, i: (bi, i, 0)),
            pl.BlockSpec((1, np_full, 2 * _COUT), lambda bi, i: (bi, 0, 0)),
        ],
        out_specs=pl.BlockSpec((1, qb, 2 * _COUT), lambda bi, i: (bi, i, 0)),
        out_shape=jax.ShapeDtypeStruct((b, nq, 2 * _COUT), jnp.float32),
    )(idxf, tsrc)


def _lxformer(xytp, features, p):
    xyt = xytp[:, :, :3]
    idx = _knn_idx(xyt, xyt, _KNN)
    h1, mean, var = _h1_stats(idx, xytp, xytp, p['pe'])
    t = features @ p['tw'].T + p['tb']
    c = _COUT
    return _branch_attn(idx, t[..., :c], t[..., c:], h1, mean, var,
                        p['pe'], p['ln_g'], p['ln_b'])


def _sparse_conv(xytp, features, cw, cb):
    b, n = xytp.shape[:2]
    pos = xytp[..., 3:4]
    neg = 1.0 - pos
    inp = jnp.concatenate([pos, neg, features], axis=-1)
    yy = jnp.clip(jnp.round(xytp[..., 2] * _H).astype(jnp.int32), 0, _H - 1)
    xx = jnp.clip(jnp.round(xytp[..., 1] * _W).astype(jnp.int32), 0, _W - 1)
    grid = jnp.zeros((b, _H, _W, inp.shape[-1]), dtype=inp.dtype)
    bidx = jnp.broadcast_to(jnp.arange(b)[:, None], (b, n))
    grid = grid.at[bidx, yy, xx].add(inp)
    out = jax.lax.conv_general_dilated(
        grid, cw, (1, 1), 'SAME',
        dimension_numbers=('NHWC', 'HWIO', 'NHWC')) + cb
    return out[bidx, yy, xx]


def _scformer(xytp, features, p):
    xyt = xytp[..., :3].at[..., 0].set(0.0)
    radius = 5.0 / _H
    idx = _select_k(xyt, xyt, _KNN, rr=radius * radius)
    xy = xytp[..., jnp.array([1, 2])]
    h1, mean, var = _h1_stats(idx, xy, xy, p['pe'])
    t = _sparse_conv(xytp, features, p['cw'], p['cb'])
    c = _COUT
    return _branch_attn(idx, t[..., :c], t[..., c:], h1, mean, var,
                        p['pe'], p['ln_g'], p['ln_b'])


def _gxformer(xytp, features, p):
    xyt = xytp[:, :, :3]
    ks = xytp.shape[1] // _R
    sample_idx = _fps(xyt, ks)
    sample_xyt = jnp.take_along_axis(xyt, sample_idx[:, :, None], axis=1)
    sample_xytp = jnp.take_along_axis(xytp, sample_idx[:, :, None], axis=1)
    pair_idx = _knn_idx(sample_xyt, xyt, _KNN)
    inv_pair_idx = _knn_idx(xyt, sample_xyt, _KNN)
    h1, mean, var = _h1_stats(inv_pair_idx, xytp, sample_xytp, p['pe'])
    t = features @ p['tw'].T + p['tb']
    c = _COUT
    pooled = _pool(pair_idx, t[..., c:])
    return _branch_attn(inv_pair_idx, t[..., :c], pooled, h1, mean, var,
                        p['pe'], p['ln_g'], p['ln_b'])


def kernel(xytp, features, params):
    lx = _lxformer(xytp, features, params['lx'])
    sc = _scformer(xytp, features, params['sc'])
    gx = _gxformer(xytp, features, params['gx'])
    h = jnp.concatenate([lx, sc, gx], axis=-1)
    h = h @ params['pw1'].T + params['pb1']
    h = jax.nn.gelu(h, approximate=False)
    return h @ params['pw2'].T + params['pb2']


# speedup vs baseline: 1.2914x; 1.0132x over previous
import math

import jax
import jax.numpy as jnp
from jax.experimental import pallas as pl

_B, _N, _CIN, _COUT, _KNN, _R, _H, _W, _FS = 4, 4096, 32, 32, 16, 8, 260, 346, 3


_SEL_QB = 128


def _make_select_kernel(K, np_full, qb, rr):
    ball = rr is not None

    def kern(q_ref, p_ref, out_ref):
        q = q_ref[0]
        pt = p_ref[0]
        dot = jax.lax.dot_general(
            q, pt, (((1,), (0,)), ((), ())),
            preferred_element_type=jnp.float32)
        q2 = jnp.sum(q * q, axis=1, keepdims=True)
        p2 = jnp.sum(pt * pt, axis=0, keepdims=True)
        d2 = q2 + p2 - 2.0 * dot
        ci = jax.lax.broadcasted_iota(jnp.int32, (qb, np_full), 1)
        cols = []
        if ball:
            rows = (jax.lax.broadcasted_iota(jnp.int32, (qb, 1), 0)
                    + pl.program_id(1) * qb)
            keys = jnp.where(d2 <= rr, ci, np_full)
            kprev = jnp.full((qb, 1), -1, jnp.int32)
            for _ in range(K):
                cand = jnp.where(keys > kprev, keys, np_full)
                m = jnp.min(cand, axis=1, keepdims=True)
                cols.append(jnp.where(m < np_full, m, rows))
                kprev = m
        else:
            mprev = jnp.full((qb, 1), -jnp.inf, jnp.float32)
            iprev = jnp.full((qb, 1), -1, jnp.int32)
            for _ in range(K):
                gt = (d2 > mprev) | ((d2 == mprev) & (ci > iprev))
                cand = jnp.where(gt, d2, jnp.inf)
                m = jnp.min(cand, axis=1, keepdims=True)
                isel = jnp.min(jnp.where(cand == m, ci, np_full), axis=1,
                               keepdims=True)
                cols.append(isel)
                mprev, iprev = m, isel
        out_ref[0] = jnp.concatenate(cols, axis=1)

    return kern


def _select_k(q, p, K, rr=None):
    b, nq, d = q.shape
    np_full = p.shape[1]
    qb = min(_SEL_QB, nq)
    pt = jnp.swapaxes(p, 1, 2)
    grid = (b, nq // qb)
    return pl.pallas_call(
        _make_select_kernel(K, np_full, qb, rr),
        grid=grid,
        in_specs=[
            pl.BlockSpec((1, qb, d), lambda bi, i: (bi, i, 0)),
            pl.BlockSpec((1, d, np_full), lambda bi, i: (bi, 0, 0)),
        ],
        out_specs=pl.BlockSpec((1, qb, K), lambda bi, i: (bi, i, 0)),
        out_shape=jax.ShapeDtypeStruct((b, nq, K), jnp.int32),
    )(q, pt)


def _knn_idx(q, p, K):
    return _select_k(q, p, K)


def _make_fps_kernel(n, ks):
    def kern(x_ref, out_ref):
        x = x_ref[0]
        ci = jax.lax.broadcasted_iota(jnp.int32, (1, n), 1)
        ck = jax.lax.broadcasted_iota(jnp.int32, (1, ks), 1)

        def body(i, st):
            dmin, last, idxs = st
            diff = x - last
            d = jnp.sum(diff * diff, axis=0, keepdims=True)
            dmin = jnp.minimum(dmin, d)
            m = jnp.max(dmin)
            nxt = jnp.min(jnp.where(dmin == m, ci, n))
            idxs = jnp.where(ck == i, nxt, idxs)
            last = jnp.sum(jnp.where(ci == nxt, x, 0.0), axis=1,
                           keepdims=True)
            return (dmin, last, idxs)

        dmin0 = jnp.full((1, n), jnp.inf, jnp.float32)
        last0 = x[:, 0:1]
        idxs0 = jnp.zeros((1, ks), jnp.int32)
        _, _, idxs = jax.lax.fori_loop(1, ks, body, (dmin0, last0, idxs0))
        out_ref[0] = idxs

    return kern


def _fps(x, K):
    b, n, _ = x.shape
    xt = jnp.swapaxes(x, 1, 2)
    xt = jnp.concatenate(
        [xt, jnp.zeros((b, 5, n), xt.dtype)], axis=1)
    return pl.pallas_call(
        _make_fps_kernel(n, K),
        grid=(b,),
        in_specs=[pl.BlockSpec((1, 8, n), lambda bi: (bi, 0, 0))],
        out_specs=pl.BlockSpec((1, 1, K), lambda bi: (bi, 0, 0)),
        out_shape=jax.ShapeDtypeStruct((b, 1, K), jnp.int32),
    )(xt)[:, 0, :]


_GB_QB = 128
_GB_CH = 512


def _onehot_gather(idx2, src, ncols, np_full):
    qbk = idx2.shape[0]
    ch = min(_GB_CH, np_full)
    acc = jnp.zeros((qbk, ncols), jnp.float32)
    for c in range(np_full // ch):
        ci = jax.lax.broadcasted_iota(jnp.int32, (qbk, ch), 1) + c * ch
        oh = (idx2 == ci).astype(jnp.float32)
        acc = acc + jax.lax.dot(oh, src[c * ch:(c + 1) * ch, :],
                                precision=jax.lax.Precision.HIGHEST,
                                preferred_element_type=jnp.float32)
    return acc


def _make_h1_stats_kernel(np_full, f):
    def kern(idxf_ref, qrep_ref, pc_ref, w1t_ref, b1_ref, h1_ref, ss_ref):
        idx2 = idxf_ref[0]
        g = _onehot_gather(idx2, pc_ref[0], f, np_full)
        diff = qrep_ref[0] - g
        h1 = jax.lax.dot(diff, w1t_ref[:],
                         preferred_element_type=jnp.float32) + b1_ref[:]
        h1_ref[0] = h1
        s0 = jnp.sum(h1, axis=0, keepdims=True)
        s1 = jnp.sum(h1 * h1, axis=0, keepdims=True)
        ss_ref[0, 0] = jnp.concatenate([s0, s1], axis=0)

    return kern


def _h1_stats(idx, qc, pc, pe):
    b, n, k = idx.shape
    np_full, f = pc.shape[1], pc.shape[2]
    nk = n * k
    qbk = _GB_QB * k
    nblk = n // _GB_QB
    idxf = idx.reshape(b, nk, 1)
    qrep = jnp.broadcast_to(qc[:, :, None, :], (b, n, k, f)).reshape(b, nk, f)
    h1, ss = pl.pallas_call(
        _make_h1_stats_kernel(np_full, f),
        grid=(b, nblk),
        in_specs=[
            pl.BlockSpec((1, qbk, 1), lambda bi, i: (bi, i, 0)),
            pl.BlockSpec((1, qbk, f), lambda bi, i: (bi, i, 0)),
            pl.BlockSpec((1, np_full, f), lambda bi, i: (bi, 0, 0)),
            pl.BlockSpec((f, f), lambda bi, i: (0, 0)),
            pl.BlockSpec((1, f), lambda bi, i: (0, 0)),
        ],
        out_specs=[
            pl.BlockSpec((1, qbk, f), lambda bi, i: (bi, i, 0)),
            pl.BlockSpec((1, 1, 2, f), lambda bi, i: (bi, i, 0, 0)),
        ],
        out_shape=[
            jax.ShapeDtypeStruct((b, nk, f), jnp.float32),
            jax.ShapeDtypeStruct((b, nblk, 2, f), jnp.float32),
        ],
    )(idxf, qrep, pc, pe['w1'].T, pe['b1'][None, :])
    s = jnp.sum(ss, axis=(0, 1))
    cnt = b * nk
    mean = s[0] / cnt
    var = s[1] / cnt - mean * mean
    return h1, mean[None, :], var[None, :]


def _make_branch_attn_kernel(np_full, f, k, qb):
    scale = math.sqrt(_COUT)
    c = _COUT

    def kern(idxf_ref, tq_ref, tsrc_ref, h1_ref, mean_ref, var_ref,
             g_ref, be_ref, w2t_ref, b2_ref, lng_ref, lnb_ref, out_ref):
        idx2 = idxf_ref[0]
        g64 = _onehot_gather(idx2, tsrc_ref[0], 2 * c, np_full)
        h1 = h1_ref[0]
        h1n = ((h1 - mean_ref[:]) * jax.lax.rsqrt(var_ref[:] + 1e-5)
               * g_ref[:] + be_ref[:])
        h1n = jnp.maximum(h1n, 0.0)
        delta = jax.lax.dot(h1n, w2t_ref[:],
                            preferred_element_type=jnp.float32) + b2_ref[:]
        psi = g64[:, :c].reshape(qb, k, c)
        alpha = g64[:, c:].reshape(qb, k, c)
        delta3 = delta.reshape(qb, k, c)
        varphi = tq_ref[0]
        x = varphi[:, None, :] - psi + delta3
        mu = jnp.mean(x, axis=-1, keepdims=True)
        xc = x - mu
        v = jnp.mean(xc * xc, axis=-1, keepdims=True)
        ln = (xc * jax.lax.rsqrt(v + 1e-5) * lng_ref[:] + lnb_ref[:]) / scale
        m = jnp.max(ln, axis=1, keepdims=True)
        e = jnp.exp(ln - m)
        a = e / jnp.sum(e, axis=1, keepdims=True)
        out_ref[0] = jnp.sum(a * (alpha + delta3), axis=1)

    return kern


def _branch_attn(idx, tq, tsrc, h1, mean, var, pe, ln_g, ln_b):
    b, n, k = idx.shape
    np_full = tsrc.shape[1]
    f = h1.shape[-1]
    c = _COUT
    qbk = _GB_QB * k
    nblk = n // _GB_QB
    idxf = idx.reshape(b, n * k, 1)
    return pl.pallas_call(
        _make_branch_attn_kernel(np_full, f, k, _GB_QB),
        grid=(b, nblk),
        in_specs=[
            pl.BlockSpec((1, qbk, 1), lambda bi, i: (bi, i, 0)),
            pl.BlockSpec((1, _GB_QB, c), lambda bi, i: (bi, i, 0)),
            pl.BlockSpec((1, np_full, 2 * c), lambda bi, i: (bi, 0, 0)),
            pl.BlockSpec((1, qbk, f), lambda bi, i: (bi, i, 0)),
            pl.BlockSpec((1, f), lambda bi, i: (0, 0)),
            pl.BlockSpec((1, f), lambda bi, i: (0, 0)),
            pl.BlockSpec((1, f), lambda bi, i: (0, 0)),
            pl.BlockSpec((1, f), lambda bi, i: (0, 0)),
            pl.BlockSpec((f, c), lambda bi, i: (0, 0)),
            pl.BlockSpec((1, c), lambda bi, i: (0, 0)),
            pl.BlockSpec((1, c), lambda bi, i: (0, 0)),
            pl.BlockSpec((1, c), lambda bi, i: (0, 0)),
        ],
        out_specs=pl.BlockSpec((1, _GB_QB, c), lambda bi, i: (bi, i, 0)),
        out_shape=jax.ShapeDtypeStruct((b, n, c), jnp.float32),
    )(idxf, tq, tsrc, h1, mean, var, pe['g'][None, :], pe['be'][None, :],
      pe['w2'].T, pe['b2'][None, :], ln_g[None, :], ln_b[None, :])


def _make_pool_kernel(np_full, k, qb):
    def kern(idxf_ref, tsrc_ref, out_ref):
        idx2 = idxf_ref[0]
        g64 = _onehot_gather(idx2, tsrc_ref[0], 2 * _COUT, np_full)
        out_ref[0] = jnp.max(g64.reshape(qb, k, 2 * _COUT), axis=1)

    return kern


def _pool(idx, tsrc):
    b, nq, k = idx.shape
    np_full = tsrc.shape[1]
    qb = min(_GB_QB, nq)
    qbk = qb * k
    idxf = idx.reshape(b, nq * k, 1)
    return pl.pallas_call(
        _make_pool_kernel(np_full, k, qb),
        grid=(b, nq // qb),
        in_specs=[
            pl.BlockSpec((1, qbk, 1), lambda bi, i: (bi, i, 0)),
            pl.BlockSpec((1, np_full, 2 * _COUT), lambda bi, i: (bi, 0, 0)),
        ],
        out_specs=pl.BlockSpec((1, qb, 2 * _COUT), lambda bi, i: (bi, i, 0)),
        out_shape=jax.ShapeDtypeStruct((b, nq, 2 * _COUT), jnp.float32),
    )(idxf, tsrc)


def _lxformer(xytp, features, p):
    xyt = xytp[:, :, :3]
    idx = _knn_idx(xyt, xyt, _KNN)
    h1, mean, var = _h1_stats(idx, xytp, xytp, p['pe'])
    t = features @ p['tw'].T + p['tb']
    c = _COUT
    return _branch_attn(idx, t[..., :c], t[..., c:], h1, mean, var,
                        p['pe'], p['ln_g'], p['ln_b'])


def _make_sparse_conv_kernel(n, cin2, qb):
    def kern(q_ref, pt_ref, inp_ref, w9_ref, cb_ref, out_ref):
        yy = q_ref[0][:, 0:1]
        xx = q_ref[0][:, 1:2]
        yyp = pt_ref[0][0:1, :]
        xxp = pt_ref[0][1:2, :]
        inp = inp_ref[0]
        acc = jnp.zeros((qb, 3 * _COUT), jnp.float32)
        for dy in (-1, 0, 1):
            for dx in (-1, 0, 1):
                match = (yyp == yy + dy) & (xxp == xx + dx)
                a = match.astype(jnp.float32)
                cell = jax.lax.dot(a, inp,
                                   precision=jax.lax.Precision.HIGHEST,
                                   preferred_element_type=jnp.float32)
                j = (dy + 1) * 3 + (dx + 1)
                acc = acc + jax.lax.dot(
                    cell, w9_ref[j * cin2:(j + 1) * cin2, :],
                    precision=jax.lax.Precision.HIGHEST,
                    preferred_element_type=jnp.float32)
        out_ref[0] = acc + cb_ref[:]

    return kern


def _sparse_conv(xytp, features, cw, cb):
    b, n = xytp.shape[:2]
    pos = xytp[..., 3:4]
    neg = 1.0 - pos
    inp = jnp.concatenate([pos, neg, features], axis=-1)
    cin2 = inp.shape[-1]
    yy = jnp.clip(jnp.round(xytp[..., 2] * _H).astype(jnp.int32), 0, _H - 1)
    xx = jnp.clip(jnp.round(xytp[..., 1] * _W).astype(jnp.int32), 0, _W - 1)
    q = jnp.stack([yy, xx], axis=-1).astype(jnp.float32)
    pt = jnp.swapaxes(q, 1, 2)
    qb = _GB_QB
    w9 = cw.reshape(9 * cin2, 3 * _COUT)
    return pl.pallas_call(
        _make_sparse_conv_kernel(n, cin2, qb),
        grid=(b, n // qb),
        in_specs=[
            pl.BlockSpec((1, qb, 2), lambda bi, i: (bi, i, 0)),
            pl.BlockSpec((1, 2, n), lambda bi, i: (bi, 0, 0)),
            pl.BlockSpec((1, n, cin2), lambda bi, i: (bi, 0, 0)),
            pl.BlockSpec((9 * cin2, 3 * _COUT), lambda bi, i: (0, 0)),
            pl.BlockSpec((1, 3 * _COUT), lambda bi, i: (0, 0)),
        ],
        out_specs=pl.BlockSpec((1, qb, 3 * _COUT), lambda bi, i: (bi, i, 0)),
        out_shape=jax.ShapeDtypeStruct((b, n, 3 * _COUT), jnp.float32),
    )(q, pt, inp, w9, cb[None, :])


def _scformer(xytp, features, p):
    xyt = xytp[..., :3].at[..., 0].set(0.0)
    radius = 5.0 / _H
    idx = _select_k(xyt, xyt, _KNN, rr=radius * radius)
    xy = xytp[..., jnp.array([1, 2])]
    h1, mean, var = _h1_stats(idx, xy, xy, p['pe'])
    t = _sparse_conv(xytp, features, p['cw'], p['cb'])
    c = _COUT
    return _branch_attn(idx, t[..., :c], t[..., c:], h1, mean, var,
                        p['pe'], p['ln_g'], p['ln_b'])


def _gxformer(xytp, features, p):
    xyt = xytp[:, :, :3]
    ks = xytp.shape[1] // _R
    sample_idx = _fps(xyt, ks)
    sample_xyt = jnp.take_along_axis(xyt, sample_idx[:, :, None], axis=1)
    sample_xytp = jnp.take_along_axis(xytp, sample_idx[:, :, None], axis=1)
    pair_idx = _knn_idx(sample_xyt, xyt, _KNN)
    inv_pair_idx = _knn_idx(xyt, sample_xyt, _KNN)
    h1, mean, var = _h1_stats(inv_pair_idx, xytp, sample_xytp, p['pe'])
    t = features @ p['tw'].T + p['tb']
    c = _COUT
    pooled = _pool(pair_idx, t[..., c:])
    return _branch_attn(inv_pair_idx, t[..., :c], pooled, h1, mean, var,
                        p['pe'], p['ln_g'], p['ln_b'])


def kernel(xytp, features, params):
    lx = _lxformer(xytp, features, params['lx'])
    sc = _scformer(xytp, features, params['sc'])
    gx = _gxformer(xytp, features, params['gx'])
    h = jnp.concatenate([lx, sc, gx], axis=-1)
    h = h @ params['pw1'].T + params['pb1']
    h = jax.nn.gelu(h, approximate=False)
    return h @ params['pw2'].T + params['pb2']


# speedup vs baseline: 1.2916x; 1.0002x over previous
import math

import jax
import jax.numpy as jnp
from jax.experimental import pallas as pl
from jax.experimental.pallas import tpu as pltpu


def _par(n):
    return pltpu.CompilerParams(dimension_semantics=("parallel",) * n)

_B, _N, _CIN, _COUT, _KNN, _R, _H, _W, _FS = 4, 4096, 32, 32, 16, 8, 260, 346, 3


_SEL_QB = 128


def _make_select_kernel(K, np_full, qb, rr):
    ball = rr is not None

    def kern(q_ref, p_ref, out_ref):
        q = q_ref[0]
        pt = p_ref[0]
        dot = jax.lax.dot_general(
            q, pt, (((1,), (0,)), ((), ())),
            preferred_element_type=jnp.float32)
        q2 = jnp.sum(q * q, axis=1, keepdims=True)
        p2 = jnp.sum(pt * pt, axis=0, keepdims=True)
        d2 = q2 + p2 - 2.0 * dot
        ci = jax.lax.broadcasted_iota(jnp.int32, (qb, np_full), 1)
        cols = []
        if ball:
            rows = (jax.lax.broadcasted_iota(jnp.int32, (qb, 1), 0)
                    + pl.program_id(1) * qb)
            keys = jnp.where(d2 <= rr, ci, np_full)
            kprev = jnp.full((qb, 1), -1, jnp.int32)
            for _ in range(K):
                cand = jnp.where(keys > kprev, keys, np_full)
                m = jnp.min(cand, axis=1, keepdims=True)
                cols.append(jnp.where(m < np_full, m, rows))
                kprev = m
        else:
            mprev = jnp.full((qb, 1), -jnp.inf, jnp.float32)
            iprev = jnp.full((qb, 1), -1, jnp.int32)
            for _ in range(K):
                gt = (d2 > mprev) | ((d2 == mprev) & (ci > iprev))
                cand = jnp.where(gt, d2, jnp.inf)
                m = jnp.min(cand, axis=1, keepdims=True)
                isel = jnp.min(jnp.where(cand == m, ci, np_full), axis=1,
                               keepdims=True)
                cols.append(isel)
                mprev, iprev = m, isel
        out_ref[0] = jnp.concatenate(cols, axis=1)

    return kern


def _select_k(q, p, K, rr=None):
    b, nq, d = q.shape
    np_full = p.shape[1]
    qb = min(_SEL_QB, nq)
    pt = jnp.swapaxes(p, 1, 2)
    grid = (b, nq // qb)
    return pl.pallas_call(
        _make_select_kernel(K, np_full, qb, rr),
        grid=grid,
        in_specs=[
            pl.BlockSpec((1, qb, d), lambda bi, i: (bi, i, 0)),
            pl.BlockSpec((1, d, np_full), lambda bi, i: (bi, 0, 0)),
        ],
        out_specs=pl.BlockSpec((1, qb, K), lambda bi, i: (bi, i, 0)),
        out_shape=jax.ShapeDtypeStruct((b, nq, K), jnp.int32),
        compiler_params=_par(2),
    )(q, pt)


def _knn_idx(q, p, K):
    return _select_k(q, p, K)


def _make_fps_kernel(n, ks):
    def kern(x_ref, out_ref):
        x = x_ref[0]
        ci = jax.lax.broadcasted_iota(jnp.int32, (1, n), 1)
        ck = jax.lax.broadcasted_iota(jnp.int32, (1, ks), 1)

        def body(i, st):
            dmin, last, idxs = st
            diff = x - last
            d = jnp.sum(diff * diff, axis=0, keepdims=True)
            dmin = jnp.minimum(dmin, d)
            m = jnp.max(dmin)
            nxt = jnp.min(jnp.where(dmin == m, ci, n))
            idxs = jnp.where(ck == i, nxt, idxs)
            last = jnp.sum(jnp.where(ci == nxt, x, 0.0), axis=1,
                           keepdims=True)
            return (dmin, last, idxs)

        dmin0 = jnp.full((1, n), jnp.inf, jnp.float32)
        last0 = x[:, 0:1]
        idxs0 = jnp.zeros((1, ks), jnp.int32)
        _, _, idxs = jax.lax.fori_loop(1, ks, body, (dmin0, last0, idxs0))
        out_ref[0] = idxs

    return kern


def _fps(x, K):
    b, n, _ = x.shape
    xt = jnp.swapaxes(x, 1, 2)
    xt = jnp.concatenate(
        [xt, jnp.zeros((b, 5, n), xt.dtype)], axis=1)
    return pl.pallas_call(
        _make_fps_kernel(n, K),
        grid=(b,),
        in_specs=[pl.BlockSpec((1, 8, n), lambda bi: (bi, 0, 0))],
        out_specs=pl.BlockSpec((1, 1, K), lambda bi: (bi, 0, 0)),
        out_shape=jax.ShapeDtypeStruct((b, 1, K), jnp.int32),
        compiler_params=_par(1),
    )(xt)[:, 0, :]


_GB_QB = 128
_GB_CH = 512


def _onehot_gather(idx2, src, ncols, np_full):
    qbk = idx2.shape[0]
    ch = min(_GB_CH, np_full)
    acc = jnp.zeros((qbk, ncols), jnp.float32)
    for c in range(np_full // ch):
        ci = jax.lax.broadcasted_iota(jnp.int32, (qbk, ch), 1) + c * ch
        oh = (idx2 == ci).astype(jnp.float32)
        acc = acc + jax.lax.dot(oh, src[c * ch:(c + 1) * ch, :],
                                precision=jax.lax.Precision.HIGHEST,
                                preferred_element_type=jnp.float32)
    return acc


def _make_h1_stats_kernel(np_full, f):
    def kern(idxf_ref, qrep_ref, pc_ref, w1t_ref, b1_ref, h1_ref, ss_ref):
        idx2 = idxf_ref[0]
        g = _onehot_gather(idx2, pc_ref[0], f, np_full)
        diff = qrep_ref[0] - g
        h1 = jax.lax.dot(diff, w1t_ref[:],
                         preferred_element_type=jnp.float32) + b1_ref[:]
        h1_ref[0] = h1
        s0 = jnp.sum(h1, axis=0, keepdims=True)
        s1 = jnp.sum(h1 * h1, axis=0, keepdims=True)
        ss_ref[0, 0] = jnp.concatenate([s0, s1], axis=0)

    return kern


def _h1_stats(idx, qc, pc, pe):
    b, n, k = idx.shape
    np_full, f = pc.shape[1], pc.shape[2]
    nk = n * k
    qbk = _GB_QB * k
    nblk = n // _GB_QB
    idxf = idx.reshape(b, nk, 1)
    qrep = jnp.broadcast_to(qc[:, :, None, :], (b, n, k, f)).reshape(b, nk, f)
    h1, ss = pl.pallas_call(
        _make_h1_stats_kernel(np_full, f),
        grid=(b, nblk),
        in_specs=[
            pl.BlockSpec((1, qbk, 1), lambda bi, i: (bi, i, 0)),
            pl.BlockSpec((1, qbk, f), lambda bi, i: (bi, i, 0)),
            pl.BlockSpec((1, np_full, f), lambda bi, i: (bi, 0, 0)),
            pl.BlockSpec((f, f), lambda bi, i: (0, 0)),
            pl.BlockSpec((1, f), lambda bi, i: (0, 0)),
        ],
        out_specs=[
            pl.BlockSpec((1, qbk, f), lambda bi, i: (bi, i, 0)),
            pl.BlockSpec((1, 1, 2, f), lambda bi, i: (bi, i, 0, 0)),
        ],
        out_shape=[
            jax.ShapeDtypeStruct((b, nk, f), jnp.float32),
            jax.ShapeDtypeStruct((b, nblk, 2, f), jnp.float32),
        ],
        compiler_params=_par(2),
    )(idxf, qrep, pc, pe['w1'].T, pe['b1'][None, :])
    s = jnp.sum(ss, axis=(0, 1))
    cnt = b * nk
    mean = s[0] / cnt
    var = s[1] / cnt - mean * mean
    return h1, mean[None, :], var[None, :]


def _make_branch_attn_kernel(np_full, f, k, qb):
    scale = math.sqrt(_COUT)
    c = _COUT

    def kern(idxf_ref, tq_ref, tsrc_ref, h1_ref, mean_ref, var_ref,
             g_ref, be_ref, w2t_ref, b2_ref, lng_ref, lnb_ref, out_ref):
        idx2 = idxf_ref[0]
        g64 = _onehot_gather(idx2, tsrc_ref[0], 2 * c, np_full)
        h1 = h1_ref[0]
        h1n = ((h1 - mean_ref[:]) * jax.lax.rsqrt(var_ref[:] + 1e-5)
               * g_ref[:] + be_ref[:])
        h1n = jnp.maximum(h1n, 0.0)
        delta = jax.lax.dot(h1n, w2t_ref[:],
                            preferred_element_type=jnp.float32) + b2_ref[:]
        psi = g64[:, :c].reshape(qb, k, c)
        alpha = g64[:, c:].reshape(qb, k, c)
        delta3 = delta.reshape(qb, k, c)
        varphi = tq_ref[0]
        x = varphi[:, None, :] - psi + delta3
        mu = jnp.mean(x, axis=-1, keepdims=True)
        xc = x - mu
        v = jnp.mean(xc * xc, axis=-1, keepdims=True)
        ln = (xc * jax.lax.rsqrt(v + 1e-5) * lng_ref[:] + lnb_ref[:]) / scale
        m = jnp.max(ln, axis=1, keepdims=True)
        e = jnp.exp(ln - m)
        a = e / jnp.sum(e, axis=1, keepdims=True)
        out_ref[0] = jnp.sum(a * (alpha + delta3), axis=1)

    return kern


def _branch_attn(idx, tq, tsrc, h1, mean, var, pe, ln_g, ln_b):
    b, n, k = idx.shape
    np_full = tsrc.shape[1]
    f = h1.shape[-1]
    c = _COUT
    qbk = _GB_QB * k
    nblk = n // _GB_QB
    idxf = idx.reshape(b, n * k, 1)
    return pl.pallas_call(
        _make_branch_attn_kernel(np_full, f, k, _GB_QB),
        grid=(b, nblk),
        in_specs=[
            pl.BlockSpec((1, qbk, 1), lambda bi, i: (bi, i, 0)),
            pl.BlockSpec((1, _GB_QB, c), lambda bi, i: (bi, i, 0)),
            pl.BlockSpec((1, np_full, 2 * c), lambda bi, i: (bi, 0, 0)),
            pl.BlockSpec((1, qbk, f), lambda bi, i: (bi, i, 0)),
            pl.BlockSpec((1, f), lambda bi, i: (0, 0)),
            pl.BlockSpec((1, f), lambda bi, i: (0, 0)),
            pl.BlockSpec((1, f), lambda bi, i: (0, 0)),
            pl.BlockSpec((1, f), lambda bi, i: (0, 0)),
            pl.BlockSpec((f, c), lambda bi, i: (0, 0)),
            pl.BlockSpec((1, c), lambda bi, i: (0, 0)),
            pl.BlockSpec((1, c), lambda bi, i: (0, 0)),
            pl.BlockSpec((1, c), lambda bi, i: (0, 0)),
        ],
        out_specs=pl.BlockSpec((1, _GB_QB, c), lambda bi, i: (bi, i, 0)),
        out_shape=jax.ShapeDtypeStruct((b, n, c), jnp.float32),
        compiler_params=_par(2),
    )(idxf, tq, tsrc, h1, mean, var, pe['g'][None, :], pe['be'][None, :],
      pe['w2'].T, pe['b2'][None, :], ln_g[None, :], ln_b[None, :])


def _make_pool_kernel(np_full, k, qb):
    def kern(idxf_ref, tsrc_ref, out_ref):
        idx2 = idxf_ref[0]
        g64 = _onehot_gather(idx2, tsrc_ref[0], 2 * _COUT, np_full)
        out_ref[0] = jnp.max(g64.reshape(qb, k, 2 * _COUT), axis=1)

    return kern


def _pool(idx, tsrc):
    b, nq, k = idx.shape
    np_full = tsrc.shape[1]
    qb = min(_GB_QB, nq)
    qbk = qb * k
    idxf = idx.reshape(b, nq * k, 1)
    return pl.pallas_call(
        _make_pool_kernel(np_full, k, qb),
        grid=(b, nq // qb),
        in_specs=[
            pl.BlockSpec((1, qbk, 1), lambda bi, i: (bi, i, 0)),
            pl.BlockSpec((1, np_full, 2 * _COUT), lambda bi, i: (bi, 0, 0)),
        ],
        out_specs=pl.BlockSpec((1, qb, 2 * _COUT), lambda bi, i: (bi, i, 0)),
        out_shape=jax.ShapeDtypeStruct((b, nq, 2 * _COUT), jnp.float32),
        compiler_params=_par(2),
    )(idxf, tsrc)


def _lxformer(xytp, features, p):
    xyt = xytp[:, :, :3]
    idx = _knn_idx(xyt, xyt, _KNN)
    h1, mean, var = _h1_stats(idx, xytp, xytp, p['pe'])
    t = features @ p['tw'].T + p['tb']
    c = _COUT
    return _branch_attn(idx, t[..., :c], t[..., c:], h1, mean, var,
                        p['pe'], p['ln_g'], p['ln_b'])


def _make_sparse_conv_kernel(n, cin2, qb):
    def kern(q_ref, pt_ref, inp_ref, w9_ref, cb_ref, out_ref):
        yy = q_ref[0][:, 0:1]
        xx = q_ref[0][:, 1:2]
        yyp = pt_ref[0][0:1, :]
        xxp = pt_ref[0][1:2, :]
        inp = inp_ref[0]
        acc = jnp.zeros((qb, 3 * _COUT), jnp.float32)
        for dy in (-1, 0, 1):
            for dx in (-1, 0, 1):
                match = (yyp == yy + dy) & (xxp == xx + dx)
                a = match.astype(jnp.float32)
                cell = jax.lax.dot(a, inp,
                                   precision=jax.lax.Precision.HIGHEST,
                                   preferred_element_type=jnp.float32)
                j = (dy + 1) * 3 + (dx + 1)
                acc = acc + jax.lax.dot(
                    cell, w9_ref[j * cin2:(j + 1) * cin2, :],
                    precision=jax.lax.Precision.HIGHEST,
                    preferred_element_type=jnp.float32)
        out_ref[0] = acc + cb_ref[:]

    return kern


def _sparse_conv(xytp, features, cw, cb):
    b, n = xytp.shape[:2]
    pos = xytp[..., 3:4]
    neg = 1.0 - pos
    inp = jnp.concatenate([pos, neg, features], axis=-1)
    cin2 = inp.shape[-1]
    yy = jnp.clip(jnp.round(xytp[..., 2] * _H).astype(jnp.int32), 0, _H - 1)
    xx = jnp.clip(jnp.round(xytp[..., 1] * _W).astype(jnp.int32), 0, _W - 1)
    q = jnp.stack([yy, xx], axis=-1).astype(jnp.float32)
    pt = jnp.swapaxes(q, 1, 2)
    qb = _GB_QB
    w9 = cw.reshape(9 * cin2, 3 * _COUT)
    return pl.pallas_call(
        _make_sparse_conv_kernel(n, cin2, qb),
        grid=(b, n // qb),
        in_specs=[
            pl.BlockSpec((1, qb, 2), lambda bi, i: (bi, i, 0)),
            pl.BlockSpec((1, 2, n), lambda bi, i: (bi, 0, 0)),
            pl.BlockSpec((1, n, cin2), lambda bi, i: (bi, 0, 0)),
            pl.BlockSpec((9 * cin2, 3 * _COUT), lambda bi, i: (0, 0)),
            pl.BlockSpec((1, 3 * _COUT), lambda bi, i: (0, 0)),
        ],
        out_specs=pl.BlockSpec((1, qb, 3 * _COUT), lambda bi, i: (bi, i, 0)),
        out_shape=jax.ShapeDtypeStruct((b, n, 3 * _COUT), jnp.float32),
        compiler_params=_par(2),
    )(q, pt, inp, w9, cb[None, :])


def _scformer(xytp, features, p):
    xyt = xytp[..., :3].at[..., 0].set(0.0)
    radius = 5.0 / _H
    idx = _select_k(xyt, xyt, _KNN, rr=radius * radius)
    xy = xytp[..., jnp.array([1, 2])]
    h1, mean, var = _h1_stats(idx, xy, xy, p['pe'])
    t = _sparse_conv(xytp, features, p['cw'], p['cb'])
    c = _COUT
    return _branch_attn(idx, t[..., :c], t[..., c:], h1, mean, var,
                        p['pe'], p['ln_g'], p['ln_b'])


def _gxformer(xytp, features, p):
    xyt = xytp[:, :, :3]
    ks = xytp.shape[1] // _R
    sample_idx = _fps(xyt, ks)
    sample_xyt = jnp.take_along_axis(xyt, sample_idx[:, :, None], axis=1)
    sample_xytp = jnp.take_along_axis(xytp, sample_idx[:, :, None], axis=1)
    pair_idx = _knn_idx(sample_xyt, xyt, _KNN)
    inv_pair_idx = _knn_idx(xyt, sample_xyt, _KNN)
    h1, mean, var = _h1_stats(inv_pair_idx, xytp, sample_xytp, p['pe'])
    t = features @ p['tw'].T + p['tb']
    c = _COUT
    pooled = _pool(pair_idx, t[..., c:])
    return _branch_attn(inv_pair_idx, t[..., :c], pooled, h1, mean, var,
                        p['pe'], p['ln_g'], p['ln_b'])


def kernel(xytp, features, params):
    lx = _lxformer(xytp, features, params['lx'])
    sc = _scformer(xytp, features, params['sc'])
    gx = _gxformer(xytp, features, params['gx'])
    h = jnp.concatenate([lx, sc, gx], axis=-1)
    h = h @ params['pw1'].T + params['pb1']
    h = jax.nn.gelu(h, approximate=False)
    return h @ params['pw2'].T + params['pb2']
